# baseline (speedup 1.0000x reference)
import functools

import jax
import jax.numpy as jnp
from jax.experimental import pallas as pl


def _listmle_body(t_row_ref, t_col_ref, s_row_ref, out_ref, *, L, CHUNK):
    t_row = t_row_ref[...].reshape(1, L)
    s_row = s_row_ref[...].reshape(1, L)
    m = jnp.max(s_row)
    sum_s = jnp.sum(s_row)
    e_row = jnp.exp(s_row - m)

    n_chunks = L // CHUNK

    def chunk_step(c, acc):
        tc = t_col_ref[0, pl.ds(c * CHUNK, CHUNK), :]
        lt = t_row < tc
        eq = t_row == tc
        ki = jax.lax.broadcasted_iota(jnp.int32, (CHUNK, L), 1)
        ii = jax.lax.broadcasted_iota(jnp.int32, (CHUNK, L), 0) + c * CHUNK
        cond = lt | (eq & (ki >= ii))
        sel = jnp.where(cond, e_row, 0.0)
        u = jnp.sum(sel, axis=1)
        return acc + jnp.sum(jnp.log(u))

    logsum = jax.lax.fori_loop(0, n_chunks, chunk_step, jnp.float32(0.0))
    loss_b = logsum / L + m - sum_s / L
    out_ref[...] = jnp.full((1, 1, 128), loss_b, dtype=jnp.float32)


def kernel(score, scope, targets_train, gpu=None):
    nb = scope.shape[0]
    L = score.shape[0] // nb
    s_row3 = score.reshape(nb, 1, L)
    t_row3 = targets_train.reshape(nb, 1, L)
    t_col3 = targets_train.reshape(nb, L, 1)

    CHUNK = 128
    body = functools.partial(_listmle_body, L=L, CHUNK=CHUNK)
    losses = pl.pallas_call(
        body,
        grid=(nb,),
        in_specs=[
            pl.BlockSpec((1, 1, L), lambda b: (b, 0, 0)),
            pl.BlockSpec((1, L, 1), lambda b: (b, 0, 0)),
            pl.BlockSpec((1, 1, L), lambda b: (b, 0, 0)),
        ],
        out_specs=pl.BlockSpec((1, 1, 128), lambda b: (b, 0, 0)),
        out_shape=jax.ShapeDtypeStruct((nb, 1, 128), jnp.float32),
    )(t_row3, t_col3, s_row3)
    return (jnp.sum(losses[:, 0, 0]) / nb).reshape(1)


# speedup vs baseline: 2.5071x; 2.5071x over previous
import functools

import jax
import jax.numpy as jnp
from jax.experimental import pallas as pl

_DOT = functools.partial(
    jax.lax.dot_general, preferred_element_type=jnp.float32)


def _listmle_body(t_row_ref, t_col_ref, s_row_ref, out_ref, *, L, C):
    n = L // C
    t_row = t_row_ref[...].reshape(1, L)
    s_row = s_row_ref[...].reshape(1, L)
    m = jnp.max(s_row)
    sum_s = jnp.sum(s_row)
    e_row = jnp.exp(s_row - m)

    eye = jnp.eye(C, dtype=jnp.float32)

    def to_col(v_row):
        return _DOT(eye, v_row, (((1,), (1,)), ((), ())))

    tri = (jax.lax.broadcasted_iota(jnp.int32, (C, C), 1)
           >= jax.lax.broadcasted_iota(jnp.int32, (C, C), 0))

    e_cols = [to_col(e_row[:, ci * C:(ci + 1) * C]) for ci in range(n)]

    u_col = [None] * n
    row_acc = [None] * n

    for ci in range(n):
        tc = t_col_ref[0, pl.ds(ci * C, C), :]
        td = t_row[:, ci * C:(ci + 1) * C]
        lt = td < tc
        eq = td == tc
        mask_d = jnp.where(lt | (eq & tri), 1.0, 0.0)
        u = _DOT(mask_d, e_cols[ci], (((1,), (0,)), ((), ())))
        if ci + 1 < n:
            w0 = (ci + 1) * C
            tr = t_row[:, w0:]
            le = jnp.where(tr <= tc, 1.0, 0.0)
            e_right = jnp.concatenate(e_cols[ci + 1:], axis=0)
            u = u + _DOT(le, e_right, (((1,), (0,)), ((), ())))
            e_ci_row = e_row[:, ci * C:(ci + 1) * C]
            sum_e_ci = jnp.sum(e_ci_row)
            contrib = sum_e_ci - _DOT(e_ci_row, le, (((1,), (0,)), ((), ())))
            for cj in range(ci + 1, n):
                piece = contrib[:, (cj - ci - 1) * C:(cj - ci) * C]
                row_acc[cj] = piece if row_acc[cj] is None else row_acc[cj] + piece
        u_col[ci] = u

    logsum = jnp.float32(0.0)
    for ci in range(n):
        u = u_col[ci]
        if row_acc[ci] is not None:
            u = u + to_col(row_acc[ci])
        logsum = logsum + jnp.sum(jnp.log(u))

    loss_b = logsum / L + m - sum_s / L
    out_ref[...] = jnp.full((1, 1, 128), loss_b, dtype=jnp.float32)


def kernel(score, scope, targets_train, gpu=None):
    nb = scope.shape[0]
    L = score.shape[0] // nb
    s_row3 = score.reshape(nb, 1, L)
    t_row3 = targets_train.reshape(nb, 1, L)
    t_col3 = targets_train.reshape(nb, L, 1)

    body = functools.partial(_listmle_body, L=L, C=128)
    losses = pl.pallas_call(
        body,
        grid=(nb,),
        in_specs=[
            pl.BlockSpec((1, 1, L), lambda b: (b, 0, 0)),
            pl.BlockSpec((1, L, 1), lambda b: (b, 0, 0)),
            pl.BlockSpec((1, 1, L), lambda b: (b, 0, 0)),
        ],
        out_specs=pl.BlockSpec((1, 1, 128), lambda b: (b, 0, 0)),
        out_shape=jax.ShapeDtypeStruct((nb, 1, 128), jnp.float32),
    )(t_row3, t_col3, s_row3)
    return (jnp.sum(losses[:, 0, 0]) / nb).reshape(1)


# speedup vs baseline: 5.3098x; 2.1179x over previous
import jax
import jax.numpy as jnp
from jax.experimental import pallas as pl
from jax.experimental.pallas import tpu as pltpu


def _xor_partner(x, dist, axis, bit_set):
    n = x.shape[axis]
    fwd = pltpu.roll(x, n - dist, axis)
    bwd = pltpu.roll(x, dist, axis)
    return jnp.where(bit_set, bwd, fwd)


def _listmle_sort_body(t_ref, s_ref, out_ref, *, NR, NC, RPS):
    t0 = t_ref[...]
    s0 = s_ref[...]
    R = jax.lax.broadcasted_iota(jnp.int32, (NR, NC), 0)
    C = jax.lax.broadcasted_iota(jnp.int32, (NR, NC), 1)
    r = R & (RPS - 1)
    idx0 = r * NC + C
    L = RPS * NC
    nseg = NR // RPS

    def do_pass(arrs, bit_set, dist, axis, kbit):
        t, ix, s = arrs
        pt = _xor_partner(t, dist, axis, bit_set)
        pix = _xor_partner(ix, dist, axis, bit_set)
        ps = _xor_partner(s, dist, axis, bit_set)
        less = (t > pt) | ((t == pt) & (ix < pix))
        up = (idx0 & kbit) == 0
        keep_min = bit_set ^ up
        take_self = ~(less ^ keep_min)
        return (jnp.where(take_self, t, pt),
                jnp.where(take_self, ix, pix),
                jnp.where(take_self, s, ps))

    def stage(k, arrs):
        kbit = jnp.left_shift(jnp.int32(1), k)

        def sub_pass(i, a):
            j = (k - 1) - i
            dr = jnp.left_shift(jnp.int32(1), j - 7)
            return do_pass(a, (R & dr) != 0, dr, 0, kbit)

        arrs = jax.lax.fori_loop(0, jnp.maximum(k - 7, 0), sub_pass, arrs)

        def lane_pass(i, a):
            j = jnp.minimum(k - 1, 6) - i
            d = jnp.left_shift(jnp.int32(1), j)
            return do_pass(a, (C & d) != 0, d, 1, kbit)

        return jax.lax.fori_loop(0, jnp.minimum(k, 7), lane_pass, arrs)

    _, _, ss = jax.lax.fori_loop(1, 12, stage, (t0, idx0, s0))

    mrow = jnp.max(s0, axis=1, keepdims=True)
    Rcol = jax.lax.broadcasted_iota(jnp.int32, (NR, 1), 0)
    for dr in (1, 2, 4, 8):
        part = _xor_partner(mrow, dr, 0, (Rcol & dr) != 0)
        mrow = jnp.maximum(mrow, part)

    y = jnp.exp(ss - mrow)
    suf = y
    for d in (1, 2, 4, 8, 16, 32, 64):
        shifted = pltpu.roll(suf, NC - d, 1)
        suf = suf + jnp.where(C + d < NC, shifted, 0.0)
    row_tot = suf[:, 0:1]
    x = row_tot
    rr = Rcol & (RPS - 1)
    for dr in (1, 2, 4, 8):
        shifted = pltpu.roll(x, NR - dr, 0)
        x = x + jnp.where(rr + dr < RPS, shifted, 0.0)
    strict = x - row_tot
    T = suf + strict
    logT = jnp.log(T)
    total = (jnp.sum(logT) / L + jnp.sum(mrow) / RPS
             - jnp.sum(s0) / L) / nseg
    out_ref[...] = jnp.full((8, 128), total, dtype=jnp.float32)


def kernel(score, scope, targets_train, gpu=None):
    nb = scope.shape[0]
    L = score.shape[0] // nb
    NC = 128
    RPS = L // NC
    NR = nb * RPS
    s2 = score.reshape(NR, NC)
    t2 = targets_train.reshape(NR, NC)

    import functools
    body = functools.partial(_listmle_sort_body, NR=NR, NC=NC, RPS=RPS)
    out = pl.pallas_call(
        body,
        out_shape=jax.ShapeDtypeStruct((8, 128), jnp.float32),
    )(t2, s2)
    return out[0, 0].reshape(1)


# speedup vs baseline: 9.0930x; 1.7125x over previous
import functools

import jax
import jax.numpy as jnp
from jax.experimental import pallas as pl
from jax.experimental.pallas import tpu as pltpu


def _partner(x, dist, axis, bit_set):
    n = x.shape[axis]
    fwd = pltpu.roll(x, n - dist, axis)
    bwd = pltpu.roll(x, dist, axis)
    return jnp.where(bit_set, bwd, fwd)


def _listmle_sort_body(t_ref, s_ref, out_ref, *, NR, NC, RPS):
    t = t_ref[...]
    s = s_ref[...]
    c_row = jax.lax.broadcasted_iota(jnp.int32, (1, NC), 1)
    r_col = jax.lax.broadcasted_iota(jnp.int32, (NR, 1), 0) & (RPS - 1)
    ix = r_col * NC + c_row
    L = RPS * NC
    nseg = NR // RPS
    s0 = s

    for k in range(1, 12):
        kbit = 1 << k
        if kbit < NC:
            up = (c_row & kbit) == 0
        elif kbit < L:
            up = (r_col & (kbit // NC)) == 0
        else:
            up = None
        for j in range(k - 1, -1, -1):
            d = 1 << j
            if d >= NC:
                dr = d // NC
                bit = (r_col & dr) != 0
                dist, axis = dr, 0
            else:
                bit = (c_row & d) != 0
                dist, axis = d, 1
            pt = _partner(t, dist, axis, bit)
            pix = _partner(ix, dist, axis, bit)
            ps = _partner(s, dist, axis, bit)
            less = (t > pt) | ((t == pt) & (ix < pix))
            keep_min = bit ^ up if up is not None else ~bit
            take_self = ~(less ^ keep_min)
            t = jnp.where(take_self, t, pt)
            ix = jnp.where(take_self, ix, pix)
            s = jnp.where(take_self, s, ps)

    mrow = jnp.max(s0, axis=1, keepdims=True)
    rbit = jax.lax.broadcasted_iota(jnp.int32, (NR, 1), 0)
    for dr in (1, 2, 4, 8):
        part = _partner(mrow, dr, 0, (rbit & dr) != 0)
        mrow = jnp.maximum(mrow, part)

    y = jnp.exp(s - mrow)
    suf = y
    for d in (1, 2, 4, 8, 16, 32, 64):
        shifted = pltpu.roll(suf, NC - d, 1)
        suf = suf + jnp.where(c_row + d < NC, shifted, 0.0)
    row_tot = suf[:, 0:1]
    x = row_tot
    for dr in (1, 2, 4, 8):
        shifted = pltpu.roll(x, NR - dr, 0)
        x = x + jnp.where(r_col + dr < RPS, shifted, 0.0)
    T = suf + (x - row_tot)
    logT = jnp.log(T)
    total = (jnp.sum(logT) / L + jnp.sum(mrow) / RPS
             - jnp.sum(s0) / L) / nseg
    out_ref[...] = jnp.full((8, 128), total, dtype=jnp.float32)


def kernel(score, scope, targets_train, gpu=None):
    nb = scope.shape[0]
    L = score.shape[0] // nb
    NC = 128
    RPS = L // NC
    NR = nb * RPS
    s2 = score.reshape(NR, NC)
    t2 = targets_train.reshape(NR, NC)

    body = functools.partial(_listmle_sort_body, NR=NR, NC=NC, RPS=RPS)
    out = pl.pallas_call(
        body,
        out_shape=jax.ShapeDtypeStruct((8, 128), jnp.float32),
    )(t2, s2)
    return out[0, 0].reshape(1)


# speedup vs baseline: 9.8028x; 1.0781x over previous
import functools

import jax
import jax.numpy as jnp
from jax.experimental import pallas as pl
from jax.experimental.pallas import tpu as pltpu


def _partner(x, dist, axis, bit_set):
    n = x.shape[axis]
    fwd = pltpu.roll(x, n - dist, axis)
    bwd = pltpu.roll(x, dist, axis)
    return jnp.where(bit_set, bwd, fwd)


def _listmle_sort_body(t_ref, s_ref, out_ref, *, NR, NC, RPS):
    t = t_ref[...]
    s0 = s_ref[...]
    c_row = jax.lax.broadcasted_iota(jnp.int32, (1, NC), 1)
    r_col = jax.lax.broadcasted_iota(jnp.int32, (NR, 1), 0) & (RPS - 1)
    ix = r_col * NC + c_row
    L = RPS * NC
    nseg = NR // RPS

    s_bf = s0.astype(jnp.bfloat16)
    s_bits = jax.lax.bitcast_convert_type(s_bf, jnp.uint16).astype(jnp.int32)
    pk = (ix << 16) | s_bits

    for k in range(1, 12):
        kbit = 1 << k
        if kbit < NC:
            up = (c_row & kbit) == 0
        elif kbit < L:
            up = (r_col & (kbit // NC)) == 0
        else:
            up = None
        for j in range(k - 1, -1, -1):
            d = 1 << j
            if d >= NC:
                dist, axis = d // NC, 0
                bit = (r_col & dist) != 0
            else:
                dist, axis = d, 1
                bit = (c_row & d) != 0
            pt = _partner(t, dist, axis, bit)
            ppk = _partner(pk, dist, axis, bit)
            less = (t > pt) | ((t == pt) & (pk < ppk))
            keep_min = bit ^ up if up is not None else ~bit
            take_self = ~(less ^ keep_min)
            t = jnp.where(take_self, t, pt)
            pk = jnp.where(take_self, pk, ppk)

    mrow = jnp.max(s0, axis=1, keepdims=True)
    rbit = jax.lax.broadcasted_iota(jnp.int32, (NR, 1), 0)
    for dr in (1, 2, 4, 8):
        part = _partner(mrow, dr, 0, (rbit & dr) != 0)
        mrow = jnp.maximum(mrow, part)

    ss = jax.lax.bitcast_convert_type(
        (pk & 0xFFFF).astype(jnp.uint16), jnp.bfloat16).astype(jnp.float32)
    y = jnp.exp(ss - mrow)
    suf = y
    for d in (1, 2, 4, 8, 16, 32, 64):
        shifted = pltpu.roll(suf, NC - d, 1)
        suf = suf + jnp.where(c_row + d < NC, shifted, 0.0)
    row_tot = suf[:, 0:1]
    x = row_tot
    for dr in (1, 2, 4, 8):
        shifted = pltpu.roll(x, NR - dr, 0)
        x = x + jnp.where(r_col + dr < RPS, shifted, 0.0)
    T = suf + (x - row_tot)
    logT = jnp.log(T)
    total = (jnp.sum(logT) / L + jnp.sum(mrow) / RPS
             - jnp.sum(s0) / L) / nseg
    out_ref[...] = jnp.full((8, 128), total, dtype=jnp.float32)


def kernel(score, scope, targets_train, gpu=None):
    nb = scope.shape[0]
    L = score.shape[0] // nb
    NC = 128
    RPS = L // NC
    NR = nb * RPS
    s2 = score.reshape(NR, NC)
    t2 = targets_train.reshape(NR, NC)

    body = functools.partial(_listmle_sort_body, NR=NR, NC=NC, RPS=RPS)
    out = pl.pallas_call(
        body,
        out_shape=jax.ShapeDtypeStruct((8, 128), jnp.float32),
    )(t2, s2)
    return out[0, 0].reshape(1)


# speedup vs baseline: 11.4538x; 1.1684x over previous
import functools

import jax
import jax.numpy as jnp
from jax.experimental import pallas as pl
from jax.experimental.pallas import tpu as pltpu


def _partner(x, dist, axis, bit_set):
    n = x.shape[axis]
    fwd = pltpu.roll(x, n - dist, axis)
    bwd = pltpu.roll(x, dist, axis)
    return jnp.where(bit_set, bwd, fwd)


def _listmle_body(t_ref, s_ref, out_ref, *, NR, NC, RPS):
    t = t_ref[...]
    s0 = s_ref[...]
    c_row = jax.lax.broadcasted_iota(jnp.int32, (1, NC), 1)
    r_col = jax.lax.broadcasted_iota(jnp.int32, (NR, 1), 0) & (RPS - 1)
    L = RPS * NC
    nseg = NR // RPS
    NB = L.bit_length() - 1

    s_bf = s0.astype(jnp.bfloat16)
    s_bits = jax.lax.bitcast_convert_type(s_bf, jnp.uint16).astype(jnp.int32)
    ix = r_col * NC + c_row
    pk = (ix << 16) | s_bits

    rbits = RPS.bit_length() - 1

    for k in range(1, NB + 1):
        kbit = 1 << k
        if k < rbits:
            up = (r_col & kbit) == 0
        elif k < NB:
            up = (c_row & (kbit >> rbits)) == 0
        else:
            up = None
        for j in range(k - 1, -1, -1):
            if j < rbits:
                dist, axis = 1 << j, 0
                bit = (r_col & dist) != 0
            else:
                dist, axis = 1 << (j - rbits), 1
                bit = (c_row & dist) != 0
            pt = _partner(t, dist, axis, bit)
            ppk = _partner(pk, dist, axis, bit)
            less = (t > pt) | ((t == pt) & (pk < ppk))
            keep_min = bit ^ up if up is not None else ~bit
            take_self = ~(less ^ keep_min)
            t = jnp.where(take_self, t, pt)
            pk = jnp.where(take_self, pk, ppk)

    mrow = jnp.max(s0, axis=1, keepdims=True)
    for dr in (1, 2, 4, 8):
        part = _partner(mrow, dr, 0, (r_col & dr) != 0)
        mrow = jnp.maximum(mrow, part)

    ss = jax.lax.bitcast_convert_type(
        (pk & 0xFFFF).astype(jnp.uint16), jnp.bfloat16).astype(jnp.float32)
    y = jnp.exp(ss - mrow)
    suf = y
    for dr in (1, 2, 4, 8):
        shifted = pltpu.roll(suf, NR - dr, 0)
        suf = suf + jnp.where(r_col + dr < RPS, shifted, 0.0)
    ct = y
    for dr in (1, 2, 4, 8):
        ct = ct + _partner(ct, dr, 0, (r_col & dr) != 0)
    hinc = ct
    for d in (1, 2, 4, 8, 16, 32, 64):
        shifted = pltpu.roll(hinc, NC - d, 1)
        hinc = hinc + jnp.where(c_row + d < NC, shifted, 0.0)
    T = suf + (hinc - ct)
    logT = jnp.log(T)
    total = (jnp.sum(logT) / L + jnp.sum(mrow) / RPS
             - jnp.sum(s0) / L) / nseg
    out_ref[...] = jnp.full((8, 128), total, dtype=jnp.float32)


def kernel(score, scope, targets_train, gpu=None):
    nb = scope.shape[0]
    L = score.shape[0] // nb
    NC = 128
    RPS = L // NC
    NR = nb * RPS
    s2 = score.reshape(NR, NC)
    t2 = targets_train.reshape(NR, NC)

    body = functools.partial(_listmle_body, NR=NR, NC=NC, RPS=RPS)
    out = pl.pallas_call(
        body,
        out_shape=jax.ShapeDtypeStruct((8, 128), jnp.float32),
    )(t2, s2)
    return out[0, 0].reshape(1)
